# baseline (speedup 1.0000x reference)
import functools

import jax
import jax.numpy as jnp
from jax import lax
from jax.experimental import pallas as pl
from jax.experimental.pallas import tpu as pltpu
from jax.experimental.pallas import tpu_sc as plsc

E = 8
TOP_K = 2
D = 768
H = 1536
T = 2048

TILE = 128
NTILES = 40
P = NTILES * TILE
CHUNK = 128

NC = 2
NS = 16
NW = NC * NS
TPW = T // NW


def _router_kernel(x_ref, wg_ref, b_ref, pp_ref, gg_ref, te_ref, loss_ref):
    x = x_ref[...]
    s = lax.dot_general(
        x, wg_ref[...], (((1,), (1,)), ((), ())),
        preferred_element_type=jnp.float32,
    ) + b_ref[...][None, :]

    idx = lax.broadcasted_iota(jnp.int32, (T, E), 1)
    m1 = jnp.max(s, axis=1, keepdims=True)
    i1 = jnp.min(jnp.where(s == m1, idx, E), axis=1, keepdims=True)
    s2 = jnp.where(idx == i1, -jnp.inf, s)
    m2 = jnp.max(s2, axis=1, keepdims=True)
    i2 = jnp.min(jnp.where(s2 == m2, idx, E), axis=1, keepdims=True)

    d = jnp.exp(m2 - m1)
    g1 = 1.0 / (1.0 + d)
    g2 = d / (1.0 + d)

    one1 = (idx == i1).astype(jnp.float32)
    one2 = (idx == i2).astype(jnp.float32)

    usage_v = jnp.sum(one1 + one2, axis=0)
    probs = jax.nn.softmax(s, axis=1)
    pmean = jnp.mean(probs, axis=0)
    usage_ratio = usage_v / jnp.sum(usage_v)
    prob_ratio = pmean / jnp.sum(pmean)
    loss_ref[0, 0] = jnp.sum(usage_ratio * prob_ratio) * E

    cnt = one1 + one2
    ii = lax.broadcasted_iota(jnp.int32, (CHUNK, CHUNK), 0)
    jj = lax.broadcasted_iota(jnp.int32, (CHUNK, CHUNK), 1)
    ltri = (jj < ii).astype(jnp.float32)
    parts = []
    run = jnp.zeros((1, E), jnp.float32)
    for c in range(T // CHUNK):
        blk = cnt[c * CHUNK:(c + 1) * CHUNK]
        within = lax.dot_general(
            ltri, blk, (((1,), (0,)), ((), ())),
            preferred_element_type=jnp.float32,
            precision=lax.Precision.HIGHEST,
        )
        parts.append(within + run)
        run = run + jnp.sum(blk, axis=0, keepdims=True)
    cexc = jnp.concatenate(parts, axis=0)
    usage = run

    pc = jnp.ceil(usage / TILE) * TILE
    er = lax.broadcasted_iota(jnp.int32, (E, E), 0)
    ec = lax.broadcasted_iota(jnp.int32, (E, E), 1)
    tri8 = (er < ec).astype(jnp.float32)
    astart = lax.dot_general(
        pc, tri8, (((1,), (0,)), ((), ())),
        preferred_element_type=jnp.float32,
        precision=lax.Precision.HIGHEST,
    )

    pos = astart + cexc
    p1 = jnp.sum(one1 * pos, axis=1)
    p2 = jnp.sum(one2 * pos, axis=1)
    pp_ref[...] = jnp.concatenate([p1[None, :], p2[None, :]], axis=0)

    lane = lax.broadcasted_iota(jnp.int32, (T, 128), 1)
    gg_ref[...] = (g1 * (lane == 0) + g2 * (lane == 1)).astype(jnp.float32)

    tj = lax.broadcasted_iota(jnp.int32, (128, E), 0).astype(jnp.float32) * TILE
    cmp = (astart <= tj).astype(jnp.float32)
    te = jnp.clip(jnp.sum(cmp, axis=1) - 1.0, 0.0, E - 1)
    te_ref[...] = te[None, :]


def _expert_kernel(te_ref, xs_ref, w1_ref, w2_ref, ys_ref):
    del te_ref
    xb = xs_ref[...].astype(jnp.bfloat16)
    w1 = w1_ref[0].astype(jnp.bfloat16)
    w2 = w2_ref[0].astype(jnp.bfloat16)
    h = lax.dot_general(
        xb, w1, (((1,), (1,)), ((), ())),
        preferred_element_type=jnp.float32,
    )
    x1 = h[:, :H]
    x2 = h[:, H:]
    act = (x1 * lax.logistic(x1) * x2).astype(jnp.bfloat16)
    ys_ref[...] = lax.dot_general(
        act, w2, (((1,), (1,)), ((), ())),
        preferred_element_type=jnp.float32,
    )


def _combine_kernel(gg_ref, y1_ref, y2_ref, out_ref):
    g1 = gg_ref[:, 0:1]
    g2 = gg_ref[:, 1:2]
    out_ref[...] = g1 * y1_ref[...] + g2 * y2_ref[...]


@functools.cache
def _sc_kernels():
    mesh = plsc.VectorSubcoreMesh(core_axis_name="c", subcore_axis_name="s")

    @functools.partial(
        pl.kernel,
        mesh=mesh,
        out_type=jax.ShapeDtypeStruct((P, D), jnp.float32),
        scratch_types=[
            pltpu.VMEM((TPW, D), jnp.float32),
            pltpu.VMEM((TPW,), jnp.int32),
            pltpu.SemaphoreType.DMA,
        ],
    )
    def sc_dispatch(x_hbm, p1_hbm, p2_hbm, xs_hbm, rows_v, idx_v, sem):
        wid = lax.axis_index("s") * NC + lax.axis_index("c")
        base = wid * TPW
        pltpu.sync_copy(x_hbm.at[pl.ds(base, TPW)], rows_v)
        pltpu.sync_copy(p1_hbm.at[pl.ds(base, TPW)], idx_v)
        pltpu.async_copy(rows_v, xs_hbm.at[idx_v], sem).wait()
        pltpu.sync_copy(p2_hbm.at[pl.ds(base, TPW)], idx_v)
        pltpu.async_copy(rows_v, xs_hbm.at[idx_v], sem).wait()

    @functools.partial(
        pl.kernel,
        mesh=mesh,
        out_type=(
            jax.ShapeDtypeStruct((T, D), jnp.float32),
            jax.ShapeDtypeStruct((T, D), jnp.float32),
        ),
        scratch_types=[
            pltpu.VMEM((TPW, D), jnp.float32),
            pltpu.VMEM((TPW,), jnp.int32),
            pltpu.SemaphoreType.DMA,
        ],
    )
    def sc_combine(ys_hbm, p1_hbm, p2_hbm, y1_hbm, y2_hbm, rows_v, idx_v, sem):
        wid = lax.axis_index("s") * NC + lax.axis_index("c")
        base = wid * TPW
        pltpu.sync_copy(p1_hbm.at[pl.ds(base, TPW)], idx_v)
        pltpu.async_copy(ys_hbm.at[idx_v], rows_v, sem).wait()
        pltpu.sync_copy(rows_v, y1_hbm.at[pl.ds(base, TPW)])
        pltpu.sync_copy(p2_hbm.at[pl.ds(base, TPW)], idx_v)
        pltpu.async_copy(ys_hbm.at[idx_v], rows_v, sem).wait()
        pltpu.sync_copy(rows_v, y2_hbm.at[pl.ds(base, TPW)])

    return sc_dispatch, sc_combine


def kernel(x, Wg, W1, W2, expert_biases):
    xf = x.reshape(T, D)

    pp_f, gg, te_f, loss = pl.pallas_call(
        _router_kernel,
        out_shape=(
            jax.ShapeDtypeStruct((2, T), jnp.float32),
            jax.ShapeDtypeStruct((T, 128), jnp.float32),
            jax.ShapeDtypeStruct((1, 128), jnp.float32),
            jax.ShapeDtypeStruct((1, 1), jnp.float32),
        ),
        in_specs=[
            pl.BlockSpec((T, D), lambda: (0, 0)),
            pl.BlockSpec((E, D), lambda: (0, 0)),
            pl.BlockSpec((E,), lambda: (0,)),
        ],
        out_specs=(
            pl.BlockSpec((2, T), lambda: (0, 0)),
            pl.BlockSpec((T, 128), lambda: (0, 0)),
            pl.BlockSpec((1, 128), lambda: (0, 0)),
            pl.BlockSpec((1, 1), lambda: (0, 0), memory_space=pltpu.SMEM),
        ),
    )(xf, Wg, expert_biases)

    p1 = pp_f[0].astype(jnp.int32)
    p2 = pp_f[1].astype(jnp.int32)
    te = te_f[0, :NTILES].astype(jnp.int32)

    sc_dispatch, sc_combine = _sc_kernels()
    xs = sc_dispatch(xf, p1, p2)

    ys = pl.pallas_call(
        _expert_kernel,
        grid_spec=pltpu.PrefetchScalarGridSpec(
            num_scalar_prefetch=1,
            grid=(NTILES,),
            in_specs=[
                pl.BlockSpec((TILE, D), lambda j, te: (j, 0)),
                pl.BlockSpec((1, 2 * H, D), lambda j, te: (te[j], 0, 0)),
                pl.BlockSpec((1, D, H), lambda j, te: (te[j], 0, 0)),
            ],
            out_specs=pl.BlockSpec((TILE, D), lambda j, te: (j, 0)),
        ),
        out_shape=jax.ShapeDtypeStruct((P, D), jnp.float32),
    )(te, xs, W1, W2)

    y1, y2 = sc_combine(ys, p1, p2)

    out = pl.pallas_call(
        _combine_kernel,
        out_shape=jax.ShapeDtypeStruct((T, D), jnp.float32),
        in_specs=[
            pl.BlockSpec((T, 128), lambda: (0, 0)),
            pl.BlockSpec((T, D), lambda: (0, 0)),
            pl.BlockSpec((T, D), lambda: (0, 0)),
        ],
        out_specs=pl.BlockSpec((T, D), lambda: (0, 0)),
    )(gg, y1, y2)

    return out.reshape(1, T, D), loss.reshape(())


# speedup vs baseline: 1.3923x; 1.3923x over previous
import functools

import jax
import jax.numpy as jnp
from jax import lax
from jax.experimental import pallas as pl
from jax.experimental.pallas import tpu as pltpu
from jax.experimental.pallas import tpu_sc as plsc

E = 8
TOP_K = 2
D = 768
H = 1536
T = 2048

TILE = 256
NTILES = 24
P = NTILES * TILE
CHUNK = 128

NC = 2
NS = 16
NW = NC * NS
TPW = T // NW


def _router_kernel(x_ref, wg_ref, b_ref, pp_ref, gg_ref, te_ref, af_ref,
                   rb_ref, loss_ref):
    x = x_ref[...]
    s = lax.dot_general(
        x, wg_ref[...], (((1,), (1,)), ((), ())),
        preferred_element_type=jnp.float32,
    ) + b_ref[...][None, :]

    idx = lax.broadcasted_iota(jnp.int32, (T, E), 1)
    m1 = jnp.max(s, axis=1, keepdims=True)
    i1 = jnp.min(jnp.where(s == m1, idx, E), axis=1, keepdims=True)
    s2 = jnp.where(idx == i1, -jnp.inf, s)
    m2 = jnp.max(s2, axis=1, keepdims=True)
    i2 = jnp.min(jnp.where(s2 == m2, idx, E), axis=1, keepdims=True)

    d = jnp.exp(m2 - m1)
    g1 = 1.0 / (1.0 + d)
    g2 = d / (1.0 + d)

    one1 = (idx == i1).astype(jnp.float32)
    one2 = (idx == i2).astype(jnp.float32)

    usage_v = jnp.sum(one1 + one2, axis=0)
    probs = jax.nn.softmax(s, axis=1)
    pmean = jnp.mean(probs, axis=0)
    usage_ratio = usage_v / jnp.sum(usage_v)
    prob_ratio = pmean / jnp.sum(pmean)
    loss_ref[0, 0] = jnp.sum(usage_ratio * prob_ratio) * E

    cnt = one1 + one2
    ii = lax.broadcasted_iota(jnp.int32, (CHUNK, CHUNK), 0)
    jj = lax.broadcasted_iota(jnp.int32, (CHUNK, CHUNK), 1)
    ltri = (jj < ii).astype(jnp.float32)
    parts = []
    run = jnp.zeros((1, E), jnp.float32)
    for c in range(T // CHUNK):
        blk = cnt[c * CHUNK:(c + 1) * CHUNK]
        within = lax.dot_general(
            ltri, blk, (((1,), (0,)), ((), ())),
            preferred_element_type=jnp.float32,
            precision=lax.Precision.HIGHEST,
        )
        parts.append(within + run)
        run = run + jnp.sum(blk, axis=0, keepdims=True)
    cexc = jnp.concatenate(parts, axis=0)
    usage = run

    pc = jnp.ceil(usage / TILE) * TILE
    er = lax.broadcasted_iota(jnp.int32, (E, E), 0)
    ec = lax.broadcasted_iota(jnp.int32, (E, E), 1)
    tri8 = (er < ec).astype(jnp.float32)
    astart = lax.dot_general(
        pc, tri8, (((1,), (0,)), ((), ())),
        preferred_element_type=jnp.float32,
        precision=lax.Precision.HIGHEST,
    )

    pos = astart + cexc
    p1 = jnp.sum(one1 * pos, axis=1)
    p2 = jnp.sum(one2 * pos, axis=1)
    pp_ref[...] = jnp.concatenate([p1[None, :], p2[None, :]], axis=0)

    lane = lax.broadcasted_iota(jnp.int32, (T, 128), 1)
    gg_ref[...] = (g1 * (lane == 0) + g2 * (lane == 1)).astype(jnp.float32)

    total = jnp.sum(pc)
    jrow = lax.broadcasted_iota(jnp.int32, (128, E), 0).astype(jnp.float32) * TILE
    tj = jnp.minimum(jrow, total - 1.0)
    cmp = (astart <= tj).astype(jnp.float32)
    te = jnp.clip(jnp.sum(cmp, axis=1) - 1.0, 0.0, E - 1)
    te_ref[...] = te[None, :]
    jt = lax.broadcasted_iota(jnp.int32, (1, 128), 1).astype(jnp.float32)
    af = (jt * TILE < total).astype(jnp.float32)
    rb = jnp.minimum(jt, total / TILE - 1.0)
    af_ref[...] = af
    rb_ref[...] = rb


def _expert_kernel(te_ref, rb_ref, af_ref, xs_ref, w1_ref, w2_ref, ys_ref):
    del te_ref, rb_ref
    j = pl.program_id(0)

    @pl.when(af_ref[j] != 0)
    def _():
        xb = xs_ref[...].astype(jnp.bfloat16)
        w1 = w1_ref[0].astype(jnp.bfloat16)
        w2 = w2_ref[0].astype(jnp.bfloat16)
        h = lax.dot_general(
            xb, w1, (((1,), (1,)), ((), ())),
            preferred_element_type=jnp.float32,
        )
        x1 = h[:, :H]
        x2 = h[:, H:]
        act = (x1 * lax.logistic(x1) * x2).astype(jnp.bfloat16)
        ys_ref[...] = lax.dot_general(
            act, w2, (((1,), (1,)), ((), ())),
            preferred_element_type=jnp.float32,
        )


def _combine_kernel(gg_ref, y1_ref, y2_ref, out_ref):
    g1 = gg_ref[:, 0:1]
    g2 = gg_ref[:, 1:2]
    out_ref[...] = g1 * y1_ref[...] + g2 * y2_ref[...]


@functools.cache
def _sc_kernels():
    mesh = plsc.VectorSubcoreMesh(core_axis_name="c", subcore_axis_name="s")

    @functools.partial(
        pl.kernel,
        mesh=mesh,
        out_type=jax.ShapeDtypeStruct((P, D), jnp.float32),
        scratch_types=[
            pltpu.VMEM((TPW, D), jnp.float32),
            pltpu.VMEM((TPW,), jnp.int32),
            pltpu.SemaphoreType.DMA,
        ],
    )
    def sc_dispatch(x_hbm, p1_hbm, p2_hbm, xs_hbm, rows_v, idx_v, sem):
        wid = lax.axis_index("s") * NC + lax.axis_index("c")
        base = wid * TPW
        pltpu.sync_copy(x_hbm.at[pl.ds(base, TPW)], rows_v)
        pltpu.sync_copy(p1_hbm.at[pl.ds(base, TPW)], idx_v)
        pltpu.async_copy(rows_v, xs_hbm.at[idx_v], sem).wait()
        pltpu.sync_copy(p2_hbm.at[pl.ds(base, TPW)], idx_v)
        pltpu.async_copy(rows_v, xs_hbm.at[idx_v], sem).wait()

    @functools.partial(
        pl.kernel,
        mesh=mesh,
        out_type=(
            jax.ShapeDtypeStruct((T, D), jnp.float32),
            jax.ShapeDtypeStruct((T, D), jnp.float32),
        ),
        scratch_types=[
            pltpu.VMEM((TPW, D), jnp.float32),
            pltpu.VMEM((TPW,), jnp.int32),
            pltpu.SemaphoreType.DMA,
        ],
    )
    def sc_combine(ys_hbm, p1_hbm, p2_hbm, y1_hbm, y2_hbm, rows_v, idx_v, sem):
        wid = lax.axis_index("s") * NC + lax.axis_index("c")
        base = wid * TPW
        pltpu.sync_copy(p1_hbm.at[pl.ds(base, TPW)], idx_v)
        pltpu.async_copy(ys_hbm.at[idx_v], rows_v, sem).wait()
        pltpu.sync_copy(rows_v, y1_hbm.at[pl.ds(base, TPW)])
        pltpu.sync_copy(p2_hbm.at[pl.ds(base, TPW)], idx_v)
        pltpu.async_copy(ys_hbm.at[idx_v], rows_v, sem).wait()
        pltpu.sync_copy(rows_v, y2_hbm.at[pl.ds(base, TPW)])

    return sc_dispatch, sc_combine


def kernel(x, Wg, W1, W2, expert_biases):
    xf = x.reshape(T, D)

    pp_f, gg, te_f, af_f, rb_f, loss = pl.pallas_call(
        _router_kernel,
        out_shape=(
            jax.ShapeDtypeStruct((2, T), jnp.float32),
            jax.ShapeDtypeStruct((T, 128), jnp.float32),
            jax.ShapeDtypeStruct((1, 128), jnp.float32),
            jax.ShapeDtypeStruct((1, 128), jnp.float32),
            jax.ShapeDtypeStruct((1, 128), jnp.float32),
            jax.ShapeDtypeStruct((1, 1), jnp.float32),
        ),
        in_specs=[
            pl.BlockSpec((T, D), lambda: (0, 0)),
            pl.BlockSpec((E, D), lambda: (0, 0)),
            pl.BlockSpec((E,), lambda: (0,)),
        ],
        out_specs=(
            pl.BlockSpec((2, T), lambda: (0, 0)),
            pl.BlockSpec((T, 128), lambda: (0, 0)),
            pl.BlockSpec((1, 128), lambda: (0, 0)),
            pl.BlockSpec((1, 128), lambda: (0, 0)),
            pl.BlockSpec((1, 128), lambda: (0, 0)),
            pl.BlockSpec((1, 1), lambda: (0, 0), memory_space=pltpu.SMEM),
        ),
    )(xf, Wg, expert_biases)

    p1 = pp_f[0].astype(jnp.int32)
    p2 = pp_f[1].astype(jnp.int32)
    te = te_f[0, :NTILES].astype(jnp.int32)
    af = af_f[0, :NTILES].astype(jnp.int32)
    rb = rb_f[0, :NTILES].astype(jnp.int32)

    sc_dispatch, sc_combine = _sc_kernels()
    xs = sc_dispatch(xf, p1, p2)

    ys = pl.pallas_call(
        _expert_kernel,
        grid_spec=pltpu.PrefetchScalarGridSpec(
            num_scalar_prefetch=3,
            grid=(NTILES,),
            in_specs=[
                pl.BlockSpec((TILE, D), lambda j, te, rb, af: (rb[j], 0)),
                pl.BlockSpec((1, 2 * H, D), lambda j, te, rb, af: (te[j], 0, 0)),
                pl.BlockSpec((1, D, H), lambda j, te, rb, af: (te[j], 0, 0)),
            ],
            out_specs=pl.BlockSpec((TILE, D), lambda j, te, rb, af: (j, 0)),
        ),
        out_shape=jax.ShapeDtypeStruct((P, D), jnp.float32),
    )(te, rb, af, xs, W1, W2)

    y1, y2 = sc_combine(ys, p1, p2)

    out = pl.pallas_call(
        _combine_kernel,
        out_shape=jax.ShapeDtypeStruct((T, D), jnp.float32),
        in_specs=[
            pl.BlockSpec((T, 128), lambda: (0, 0)),
            pl.BlockSpec((T, D), lambda: (0, 0)),
            pl.BlockSpec((T, D), lambda: (0, 0)),
        ],
        out_specs=pl.BlockSpec((T, D), lambda: (0, 0)),
    )(gg, y1, y2)

    return out.reshape(1, T, D), loss.reshape(())
